# Initial kernel scaffold; baseline (speedup 1.0000x reference)
import jax
import jax.numpy as jnp
from jax.experimental import pallas as pl


def kernel(h, edge_index, edge_weight, W, b, gamma, beta):
    raise NotImplementedError("write your pallas kernel here")


# speedup vs baseline: 6.7618x; 6.7618x over previous
import functools

import jax
import jax.numpy as jnp
from jax import lax
from jax.experimental import pallas as pl
from jax.experimental.pallas import tpu as pltpu
from jax.experimental.pallas import tpu_sc as plsc

N = 10000
E = 320000
D = 128
L = 16
NC = 2
NS = 16
NW = NC * NS
NP = 10240
STRIPE = NP // NS
E_PER_W = E // NW
CPT = D // NW
CHUNK = 2000


def _rsqrt_f32(x):
    xi = lax.bitcast_convert_type(x, jnp.int32)
    yi = 0x5F3759DF - lax.shift_right_arithmetic(xi, 1)
    y = lax.bitcast_convert_type(yi, jnp.float32)
    hx = 0.5 * x
    for _ in range(3):
        y = y * (1.5 - hx * y * y)
    return y


def _deg_kernel(dst_hbm, ew_hbm, degp_hbm, dst_v, ew_v, deg_l, shared, red_v,
                out_v):
    cid = lax.axis_index("c")
    sid = lax.axis_index("s")
    wid = cid * NS + sid

    zero16 = jnp.zeros((L,), jnp.float32)

    @pl.loop(0, NP // L)
    def _(i):
        deg_l[pl.ds(i * L, L)] = zero16

    pltpu.sync_copy(dst_hbm.at[wid], dst_v)
    pltpu.sync_copy(ew_hbm.at[wid], ew_v)

    @pl.loop(0, E_PER_W // L)
    def _(g):
        dv = dst_v[pl.ds(g * L, L)]
        wv = ew_v[pl.ds(g * L, L)]
        plsc.addupdate_scatter(deg_l, [dv], wv)

    pltpu.sync_copy(deg_l, shared.at[sid])
    plsc.subcore_barrier()

    base = sid * STRIPE
    pltpu.sync_copy(shared.at[:, pl.ds(base, STRIPE)], red_v)

    @pl.loop(0, STRIPE // L)
    def _(g):
        acc = red_v[0, pl.ds(g * L, L)]
        for t in range(1, NS):
            acc = acc + red_v[t, pl.ds(g * L, L)]
        out_v[pl.ds(g * L, L)] = acc

    pltpu.sync_copy(out_v, degp_hbm.at[cid, pl.ds(base, STRIPE)])


_deg_call = pl.kernel(
    _deg_kernel,
    out_type=jax.ShapeDtypeStruct((NC, NP), jnp.float32),
    mesh=plsc.VectorSubcoreMesh(core_axis_name="c", subcore_axis_name="s"),
    compiler_params=pltpu.CompilerParams(needs_layout_passes=False),
    scratch_types=[
        pltpu.VMEM((E_PER_W,), jnp.int32),
        pltpu.VMEM((E_PER_W,), jnp.float32),
        pltpu.VMEM((NP,), jnp.float32),
        pltpu.VMEM_SHARED((NS, NP), jnp.float32),
        pltpu.VMEM((NS, STRIPE), jnp.float32),
        pltpu.VMEM((STRIPE,), jnp.float32),
    ],
)


def _agg_kernel(degp_hbm, ht_hbm, src_hbm, dst_hbm, ew_hbm, agg_hbm,
                degb, d_v, hc, agg, sbuf, tbuf, wbuf):
    cid = lax.axis_index("c")
    sid = lax.axis_index("s")
    wid = cid * NS + sid

    pltpu.sync_copy(degp_hbm, degb)

    @pl.loop(0, NP // L)
    def _(g):
        sl = pl.ds(g * L, L)
        deg = degb[0, sl] + degb[1, sl] + 1.0
        d_v[sl] = jnp.minimum(_rsqrt_f32(deg), 10000.0)

    pltpu.sync_copy(ht_hbm.at[pl.ds(wid * CPT, CPT)], hc)

    @pl.loop(0, NP // L)
    def _(g):
        sl = pl.ds(g * L, L)
        dv = d_v[sl]
        d2 = dv * dv
        for c in range(CPT):
            agg[c, sl] = hc[c, sl] * d2

    @pl.loop(0, E // CHUNK)
    def _(ch):
        off = ch * CHUNK
        pltpu.sync_copy(src_hbm.at[pl.ds(off, CHUNK)], sbuf)
        pltpu.sync_copy(dst_hbm.at[pl.ds(off, CHUNK)], tbuf)
        pltpu.sync_copy(ew_hbm.at[pl.ds(off, CHUNK)], wbuf)

        @pl.loop(0, CHUNK // L)
        def _(g):
            sl = pl.ds(g * L, L)
            s_v = sbuf[sl]
            t_v = tbuf[sl]
            w_v = wbuf[sl]
            norm = plsc.load_gather(d_v, [s_v]) * w_v \
                * plsc.load_gather(d_v, [t_v])
            for c in range(CPT):
                cv = jnp.full((L,), c, jnp.int32)
                hv = plsc.load_gather(hc, [cv, s_v])
                plsc.addupdate_scatter(agg, [cv, t_v], hv * norm)

    pltpu.sync_copy(agg, agg_hbm.at[pl.ds(wid * CPT, CPT)])


_agg_call = pl.kernel(
    _agg_kernel,
    out_type=jax.ShapeDtypeStruct((D, NP), jnp.float32),
    mesh=plsc.VectorSubcoreMesh(core_axis_name="c", subcore_axis_name="s"),
    compiler_params=pltpu.CompilerParams(needs_layout_passes=False),
    scratch_types=[
        pltpu.VMEM((NC, NP), jnp.float32),
        pltpu.VMEM((NP,), jnp.float32),
        pltpu.VMEM((CPT, NP), jnp.float32),
        pltpu.VMEM((CPT, NP), jnp.float32),
        pltpu.VMEM((CHUNK,), jnp.int32),
        pltpu.VMEM((CHUNK,), jnp.int32),
        pltpu.VMEM((CHUNK,), jnp.float32),
    ],
)


BN = 1024


def _dense_kernel(aggt_ref, h_ref, w_ref, b_ref, g_ref, be_ref, o_ref):
    agg_t = aggt_ref[...]
    x = lax.dot_general(agg_t, w_ref[...], (((0,), (1,)), ((), ())),
                        preferred_element_type=jnp.float32)
    x = x + b_ref[...] + h_ref[...]
    mean = jnp.mean(x, axis=1, keepdims=True)
    xc = x - mean
    var = jnp.mean(xc * xc, axis=1, keepdims=True)
    xn = xc * lax.rsqrt(var + 1e-5)
    x = xn * g_ref[...] + be_ref[...]
    o_ref[...] = 0.5 * x * (1.0 + lax.erf(x * 0.7071067811865476))


def _dense_call(agg_t, h, W, b, gamma, beta):
    return pl.pallas_call(
        _dense_kernel,
        grid=(NP // BN,),
        in_specs=[
            pl.BlockSpec((D, BN), lambda i: (0, i)),
            pl.BlockSpec((BN, D), lambda i: (i, 0)),
            pl.BlockSpec((D, D), lambda i: (0, 0)),
            pl.BlockSpec((1, D), lambda i: (0, 0)),
            pl.BlockSpec((1, D), lambda i: (0, 0)),
            pl.BlockSpec((1, D), lambda i: (0, 0)),
        ],
        out_specs=pl.BlockSpec((BN, D), lambda i: (i, 0)),
        out_shape=jax.ShapeDtypeStruct((NP, D), jnp.float32),
    )(agg_t, h, W, b, gamma, beta)


def kernel(h, edge_index, edge_weight, W, b, gamma, beta):
    src = edge_index[0].astype(jnp.int32)
    dst = edge_index[1].astype(jnp.int32)
    ew = edge_weight.astype(jnp.float32)
    h = h.astype(jnp.float32)
    h_t = jnp.pad(h.T, ((0, 0), (0, NP - N)))
    h_p = jnp.pad(h, ((0, NP - N), (0, 0)))

    degp = _deg_call(dst.reshape(NW, E_PER_W), ew.reshape(NW, E_PER_W))
    agg_t = _agg_call(degp, h_t, src, dst, ew)
    out = _dense_call(agg_t, h_p, W, b.reshape(1, D), gamma.reshape(1, D),
                      beta.reshape(1, D))
    return out[:N]


# speedup vs baseline: 12.6123x; 1.8652x over previous
import jax
import jax.numpy as jnp
from jax import lax
from jax.experimental import pallas as pl
from jax.experimental.pallas import tpu as pltpu
from jax.experimental.pallas import tpu_sc as plsc

N = 10000
E = 320000
D = 128
L = 16
NC = 2
NS = 16
NW = NC * NS
NP = 10240
STRIPE = NP // NS
E_PER_W = E // NW
CPT = D // NW
CHUNK = 8000
SMASK = 0x3FFF


def _rsqrt_f32(x):
    xi = lax.bitcast_convert_type(x, jnp.int32)
    yi = 0x5F3759DF - lax.shift_right_arithmetic(xi, 1)
    y = lax.bitcast_convert_type(yi, jnp.float32)
    hx = 0.5 * x
    for _ in range(3):
        y = y * (1.5 - hx * y * y)
    return y


def _build_d(degb, d_v):
    @pl.loop(0, NP // L)
    def _(g):
        sl = pl.ds(g * L, L)
        deg = degb[0, sl] + degb[1, sl] + 1.0
        d_v[sl] = jnp.minimum(_rsqrt_f32(deg), 10000.0)


def _deg_kernel(dst_hbm, ew_hbm, degp_hbm, dst_v, ew_v, deg_l, shared, red_v,
                out_v):
    cid = lax.axis_index("c")
    sid = lax.axis_index("s")
    wid = cid * NS + sid

    zero16 = jnp.zeros((L,), jnp.float32)

    @pl.loop(0, NP // L)
    def _(i):
        deg_l[pl.ds(i * L, L)] = zero16

    pltpu.sync_copy(dst_hbm.at[wid], dst_v)
    pltpu.sync_copy(ew_hbm.at[wid], ew_v)

    @pl.loop(0, E_PER_W // L, unroll=4)
    def _(g):
        dv = dst_v[pl.ds(g * L, L)]
        wv = ew_v[pl.ds(g * L, L)]
        plsc.addupdate_scatter(deg_l, [dv], wv)

    pltpu.sync_copy(deg_l, shared.at[sid])
    plsc.subcore_barrier()

    base = sid * STRIPE
    pltpu.sync_copy(shared.at[:, pl.ds(base, STRIPE)], red_v)

    @pl.loop(0, STRIPE // L)
    def _(g):
        acc = red_v[0, pl.ds(g * L, L)]
        for t in range(1, NS):
            acc = acc + red_v[t, pl.ds(g * L, L)]
        out_v[pl.ds(g * L, L)] = acc

    pltpu.sync_copy(out_v, degp_hbm.at[cid, pl.ds(base, STRIPE)])


_deg_call = pl.kernel(
    _deg_kernel,
    out_type=jax.ShapeDtypeStruct((NC, NP), jnp.float32),
    mesh=plsc.VectorSubcoreMesh(core_axis_name="c", subcore_axis_name="s"),
    compiler_params=pltpu.CompilerParams(needs_layout_passes=False),
    scratch_types=[
        pltpu.VMEM((E_PER_W,), jnp.int32),
        pltpu.VMEM((E_PER_W,), jnp.float32),
        pltpu.VMEM((NP,), jnp.float32),
        pltpu.VMEM_SHARED((NS, NP), jnp.float32),
        pltpu.VMEM((NS, STRIPE), jnp.float32),
        pltpu.VMEM((STRIPE,), jnp.float32),
    ],
)


D2_STRIPE = NP // NW


def _norm_kernel(degp_hbm, spk_hbm, ew_hbm, norm_hbm, d2_hbm,
                 degb, d_v, spk_v, ew_v, nrm_v, d2_v):
    cid = lax.axis_index("c")
    sid = lax.axis_index("s")
    wid = cid * NS + sid

    pltpu.sync_copy(degp_hbm, degb)
    _build_d(degb, d_v)

    base = wid * D2_STRIPE

    @pl.loop(0, D2_STRIPE // L)
    def _(g):
        dv = d_v[pl.ds(base + g * L, L)]
        d2_v[pl.ds(g * L, L)] = dv * dv

    pltpu.sync_copy(d2_v, d2_hbm.at[pl.ds(base, D2_STRIPE)])

    pltpu.sync_copy(spk_hbm.at[pl.ds(wid * E_PER_W, E_PER_W)], spk_v)
    pltpu.sync_copy(ew_hbm.at[pl.ds(wid * E_PER_W, E_PER_W)], ew_v)

    @pl.loop(0, E_PER_W // L, unroll=4)
    def _(g):
        sl = pl.ds(g * L, L)
        p = spk_v[sl]
        s_v = p & SMASK
        t_v = lax.shift_right_logical(p, 14)
        nrm_v[sl] = plsc.load_gather(d_v, [s_v]) * ew_v[sl] \
            * plsc.load_gather(d_v, [t_v])

    pltpu.sync_copy(nrm_v, norm_hbm.at[pl.ds(wid * E_PER_W, E_PER_W)])


_norm_call = pl.kernel(
    _norm_kernel,
    out_type=(jax.ShapeDtypeStruct((E,), jnp.float32),
              jax.ShapeDtypeStruct((NP,), jnp.float32)),
    mesh=plsc.VectorSubcoreMesh(core_axis_name="c", subcore_axis_name="s"),
    compiler_params=pltpu.CompilerParams(needs_layout_passes=False),
    scratch_types=[
        pltpu.VMEM((NC, NP), jnp.float32),
        pltpu.VMEM((NP,), jnp.float32),
        pltpu.VMEM((E_PER_W,), jnp.int32),
        pltpu.VMEM((E_PER_W,), jnp.float32),
        pltpu.VMEM((E_PER_W,), jnp.float32),
        pltpu.VMEM((D2_STRIPE,), jnp.float32),
    ],
)


HMASK = -65536


def _unpack2(p):
    lo = lax.bitcast_convert_type(lax.shift_left(p, 16), jnp.float32)
    hi = lax.bitcast_convert_type(p & HMASK, jnp.float32)
    return lo, hi


def _agg_kernel(hpk_hbm, spk_hbm, nrm_hbm, d2_hbm, agg_hbm,
                hcp, agg, d2_v, spk_b, nrm_b):
    cid = lax.axis_index("c")
    sid = lax.axis_index("s")
    wid = cid * NS + sid

    pltpu.sync_copy(hpk_hbm.at[pl.ds(wid * 2, 2)], hcp)
    pltpu.sync_copy(d2_hbm, d2_v)

    @pl.loop(0, NP // L)
    def _(g):
        sl = pl.ds(g * L, L)
        d2 = d2_v[sl]
        c0, c1 = _unpack2(hcp[0, sl])
        c2, c3 = _unpack2(hcp[1, sl])
        agg[0, sl] = c0 * d2
        agg[1, sl] = c1 * d2
        agg[2, sl] = c2 * d2
        agg[3, sl] = c3 * d2

    row0 = jnp.zeros((L,), jnp.int32)
    row1 = jnp.full((L,), 1, jnp.int32)
    cv0 = jnp.zeros((L,), jnp.int32)
    cv1 = jnp.full((L,), 1, jnp.int32)
    cv2 = jnp.full((L,), 2, jnp.int32)
    cv3 = jnp.full((L,), 3, jnp.int32)

    @pl.loop(0, E // CHUNK)
    def _(ch):
        off = ch * CHUNK
        pltpu.sync_copy(spk_hbm.at[pl.ds(off, CHUNK)], spk_b)
        pltpu.sync_copy(nrm_hbm.at[pl.ds(off, CHUNK)], nrm_b)

        @pl.loop(0, CHUNK // L, unroll=4)
        def _(g):
            sl = pl.ds(g * L, L)
            p = spk_b[sl]
            s_v = p & SMASK
            t_v = lax.shift_right_logical(p, 14)
            nv = nrm_b[sl]
            g0 = plsc.load_gather(hcp, [row0, s_v])
            g1 = plsc.load_gather(hcp, [row1, s_v])
            c0, c1 = _unpack2(g0)
            c2, c3 = _unpack2(g1)
            plsc.addupdate_scatter(agg, [cv0, t_v], c0 * nv)
            plsc.addupdate_scatter(agg, [cv1, t_v], c1 * nv)
            plsc.addupdate_scatter(agg, [cv2, t_v], c2 * nv)
            plsc.addupdate_scatter(agg, [cv3, t_v], c3 * nv)

    pltpu.sync_copy(agg, agg_hbm.at[pl.ds(wid * CPT, CPT)])


_agg_call = pl.kernel(
    _agg_kernel,
    out_type=jax.ShapeDtypeStruct((D, NP), jnp.float32),
    mesh=plsc.VectorSubcoreMesh(core_axis_name="c", subcore_axis_name="s"),
    compiler_params=pltpu.CompilerParams(needs_layout_passes=False),
    scratch_types=[
        pltpu.VMEM((2, NP), jnp.int32),
        pltpu.VMEM((CPT, NP), jnp.float32),
        pltpu.VMEM((NP,), jnp.float32),
        pltpu.VMEM((CHUNK,), jnp.int32),
        pltpu.VMEM((CHUNK,), jnp.float32),
    ],
)


BN = 1024


def _dense_kernel(aggt_ref, h_ref, w_ref, b_ref, g_ref, be_ref, o_ref):
    agg_t = aggt_ref[...]
    x = lax.dot_general(agg_t, w_ref[...], (((0,), (1,)), ((), ())),
                        preferred_element_type=jnp.float32)
    x = x + b_ref[...] + h_ref[...]
    mean = jnp.mean(x, axis=1, keepdims=True)
    xc = x - mean
    var = jnp.mean(xc * xc, axis=1, keepdims=True)
    xn = xc * lax.rsqrt(var + 1e-5)
    x = xn * g_ref[...] + be_ref[...]
    o_ref[...] = 0.5 * x * (1.0 + lax.erf(x * 0.7071067811865476))


def _dense_call(agg_t, h, W, b, gamma, beta):
    return pl.pallas_call(
        _dense_kernel,
        grid=(NP // BN,),
        in_specs=[
            pl.BlockSpec((D, BN), lambda i: (0, i)),
            pl.BlockSpec((BN, D), lambda i: (i, 0)),
            pl.BlockSpec((D, D), lambda i: (0, 0)),
            pl.BlockSpec((1, D), lambda i: (0, 0)),
            pl.BlockSpec((1, D), lambda i: (0, 0)),
            pl.BlockSpec((1, D), lambda i: (0, 0)),
        ],
        out_specs=pl.BlockSpec((BN, D), lambda i: (i, 0)),
        out_shape=jax.ShapeDtypeStruct((NP, D), jnp.float32),
    )(agg_t, h, W, b, gamma, beta)


def kernel(h, edge_index, edge_weight, W, b, gamma, beta):
    src = edge_index[0].astype(jnp.int32)
    dst = edge_index[1].astype(jnp.int32)
    ew = edge_weight.astype(jnp.float32)
    h = h.astype(jnp.float32)
    h_p = jnp.pad(h, ((0, NP - N), (0, 0)))

    hb = lax.bitcast_convert_type(
        jnp.pad(h.T, ((0, 0), (0, NP - N))).astype(jnp.bfloat16), jnp.uint16
    ).astype(jnp.uint32)
    hpk = lax.bitcast_convert_type(hb[0::2] | (hb[1::2] << 16), jnp.int32)

    spk = src | (dst << 14)

    degp = _deg_call(dst.reshape(NW, E_PER_W), ew.reshape(NW, E_PER_W))
    norm, d2 = _norm_call(degp, spk, ew)
    agg_t = _agg_call(hpk, spk, norm, d2)
    out = _dense_call(agg_t, h_p, W, b.reshape(1, D), gamma.reshape(1, D),
                      beta.reshape(1, D))
    return out[:N]


# speedup vs baseline: 13.0972x; 1.0384x over previous
import jax
import jax.numpy as jnp
from jax import lax
from jax.experimental import pallas as pl
from jax.experimental.pallas import tpu as pltpu
from jax.experimental.pallas import tpu_sc as plsc

N = 10000
E = 320000
D = 128
L = 16
NC = 2
NS = 16
NW = NC * NS
NP = 10240
STRIPE = NP // NS
E_PER_W = E // NW
CPT = D // NW
CHUNK = 8000
SMASK = 0x3FFF


def _rsqrt_f32(x):
    xi = lax.bitcast_convert_type(x, jnp.int32)
    yi = 0x5F3759DF - lax.shift_right_arithmetic(xi, 1)
    y = lax.bitcast_convert_type(yi, jnp.float32)
    hx = 0.5 * x
    for _ in range(3):
        y = y * (1.5 - hx * y * y)
    return y


def _build_d(degb, d_v):
    @pl.loop(0, NP // L)
    def _(g):
        sl = pl.ds(g * L, L)
        deg = degb[0, sl] + degb[1, sl] + 1.0
        d_v[sl] = jnp.minimum(_rsqrt_f32(deg), 10000.0)


def _deg_kernel(dst_hbm, ew_hbm, degp_hbm, dst_v, ew_v, deg_l, shared, red_v,
                out_v):
    cid = lax.axis_index("c")
    sid = lax.axis_index("s")
    wid = cid * NS + sid

    zero16 = jnp.zeros((L,), jnp.float32)

    @pl.loop(0, NP // L)
    def _(i):
        deg_l[pl.ds(i * L, L)] = zero16

    pltpu.sync_copy(dst_hbm.at[wid], dst_v)
    pltpu.sync_copy(ew_hbm.at[wid], ew_v)

    @pl.loop(0, E_PER_W // L, unroll=4)
    def _(g):
        dv = dst_v[pl.ds(g * L, L)]
        wv = ew_v[pl.ds(g * L, L)]
        plsc.addupdate_scatter(deg_l, [dv], wv)

    pltpu.sync_copy(deg_l, shared.at[sid])
    plsc.subcore_barrier()

    base = sid * STRIPE
    pltpu.sync_copy(shared.at[:, pl.ds(base, STRIPE)], red_v)

    @pl.loop(0, STRIPE // L)
    def _(g):
        acc = red_v[0, pl.ds(g * L, L)]
        for t in range(1, NS):
            acc = acc + red_v[t, pl.ds(g * L, L)]
        out_v[pl.ds(g * L, L)] = acc

    pltpu.sync_copy(out_v, degp_hbm.at[cid, pl.ds(base, STRIPE)])


_deg_call = pl.kernel(
    _deg_kernel,
    out_type=jax.ShapeDtypeStruct((NC, NP), jnp.float32),
    mesh=plsc.VectorSubcoreMesh(core_axis_name="c", subcore_axis_name="s"),
    compiler_params=pltpu.CompilerParams(needs_layout_passes=False),
    scratch_types=[
        pltpu.VMEM((E_PER_W,), jnp.int32),
        pltpu.VMEM((E_PER_W,), jnp.float32),
        pltpu.VMEM((NP,), jnp.float32),
        pltpu.VMEM_SHARED((NS, NP), jnp.float32),
        pltpu.VMEM((NS, STRIPE), jnp.float32),
        pltpu.VMEM((STRIPE,), jnp.float32),
    ],
)


D2_STRIPE = NP // NW


def _norm_kernel(degp_hbm, spk_hbm, ew_hbm, norm_hbm, d2_hbm,
                 degb, d_v, spk_v, ew_v, nrm_v, d2_v):
    cid = lax.axis_index("c")
    sid = lax.axis_index("s")
    wid = cid * NS + sid

    pltpu.sync_copy(degp_hbm, degb)
    _build_d(degb, d_v)

    base = wid * D2_STRIPE

    @pl.loop(0, D2_STRIPE // L)
    def _(g):
        dv = d_v[pl.ds(base + g * L, L)]
        d2_v[pl.ds(g * L, L)] = dv * dv

    pltpu.sync_copy(d2_v, d2_hbm.at[pl.ds(base, D2_STRIPE)])

    pltpu.sync_copy(spk_hbm.at[pl.ds(wid * E_PER_W, E_PER_W)], spk_v)
    pltpu.sync_copy(ew_hbm.at[pl.ds(wid * E_PER_W, E_PER_W)], ew_v)

    @pl.loop(0, E_PER_W // L, unroll=4)
    def _(g):
        sl = pl.ds(g * L, L)
        p = spk_v[sl]
        s_v = p & SMASK
        t_v = lax.shift_right_logical(p, 14)
        nrm_v[sl] = plsc.load_gather(d_v, [s_v]) * ew_v[sl] \
            * plsc.load_gather(d_v, [t_v])

    pltpu.sync_copy(nrm_v, norm_hbm.at[pl.ds(wid * E_PER_W, E_PER_W)])


_norm_call = pl.kernel(
    _norm_kernel,
    out_type=(jax.ShapeDtypeStruct((E,), jnp.float32),
              jax.ShapeDtypeStruct((NP,), jnp.float32)),
    mesh=plsc.VectorSubcoreMesh(core_axis_name="c", subcore_axis_name="s"),
    compiler_params=pltpu.CompilerParams(needs_layout_passes=False),
    scratch_types=[
        pltpu.VMEM((NC, NP), jnp.float32),
        pltpu.VMEM((NP,), jnp.float32),
        pltpu.VMEM((E_PER_W,), jnp.int32),
        pltpu.VMEM((E_PER_W,), jnp.float32),
        pltpu.VMEM((E_PER_W,), jnp.float32),
        pltpu.VMEM((D2_STRIPE,), jnp.float32),
    ],
)


HMASK = -65536


def _unpack2(p):
    lo = lax.bitcast_convert_type(lax.shift_left(p, 16), jnp.float32)
    hi = lax.bitcast_convert_type(p & HMASK, jnp.float32)
    return lo, hi


def _agg_kernel(hpk_hbm, spk_hbm, nrm_hbm, d2_hbm, agg_hbm,
                hcp, agg, d2_v, spk_b, nrm_b):
    cid = lax.axis_index("c")
    sid = lax.axis_index("s")
    wid = cid * NS + sid

    pltpu.sync_copy(hpk_hbm.at[pl.ds(wid * 2 * NP, 2 * NP)], hcp)
    pltpu.sync_copy(d2_hbm, d2_v)

    @pl.loop(0, NP // L)
    def _(g):
        d2 = d2_v[pl.ds(g * L, L)]
        c0, c1 = _unpack2(hcp[pl.ds(g * L, L)])
        c2, c3 = _unpack2(hcp[pl.ds(NP + g * L, L)])
        agg[pl.ds(g * L, L)] = c0 * d2
        agg[pl.ds(NP + g * L, L)] = c1 * d2
        agg[pl.ds(2 * NP + g * L, L)] = c2 * d2
        agg[pl.ds(3 * NP + g * L, L)] = c3 * d2

    @pl.loop(0, E // CHUNK)
    def _(ch):
        off = ch * CHUNK
        pltpu.sync_copy(spk_hbm.at[pl.ds(off, CHUNK)], spk_b)
        pltpu.sync_copy(nrm_hbm.at[pl.ds(off, CHUNK)], nrm_b)

        @plsc.parallel_loop(0, CHUNK // L, unroll=8)
        def _(g):
            sl = pl.ds(g * L, L)
            p = spk_b[sl]
            s_v = p & SMASK
            t_v = lax.shift_right_logical(p, 14)
            nv = nrm_b[sl]
            g0 = plsc.load_gather(hcp, [s_v])
            g1 = plsc.load_gather(hcp, [s_v + NP])
            c0, c1 = _unpack2(g0)
            c2, c3 = _unpack2(g1)
            t1 = t_v + NP
            t2 = t1 + NP
            t3 = t2 + NP
            plsc.addupdate_scatter(agg, [t_v], c0 * nv)
            plsc.addupdate_scatter(agg, [t1], c1 * nv)
            plsc.addupdate_scatter(agg, [t2], c2 * nv)
            plsc.addupdate_scatter(agg, [t3], c3 * nv)

    pltpu.sync_copy(agg, agg_hbm.at[pl.ds(wid * CPT * NP, CPT * NP)])


_agg_call = pl.kernel(
    _agg_kernel,
    out_type=jax.ShapeDtypeStruct((D * NP,), jnp.float32),
    mesh=plsc.VectorSubcoreMesh(core_axis_name="c", subcore_axis_name="s"),
    compiler_params=pltpu.CompilerParams(needs_layout_passes=False),
    scratch_types=[
        pltpu.VMEM((2 * NP,), jnp.int32),
        pltpu.VMEM((CPT * NP,), jnp.float32),
        pltpu.VMEM((NP,), jnp.float32),
        pltpu.VMEM((CHUNK,), jnp.int32),
        pltpu.VMEM((CHUNK,), jnp.float32),
    ],
)


BN = 1024


def _dense_kernel(aggt_ref, h_ref, w_ref, b_ref, g_ref, be_ref, o_ref):
    agg_t = aggt_ref[...]
    x = lax.dot_general(agg_t, w_ref[...], (((0,), (1,)), ((), ())),
                        preferred_element_type=jnp.float32)
    x = x + b_ref[...] + h_ref[...]
    mean = jnp.mean(x, axis=1, keepdims=True)
    xc = x - mean
    var = jnp.mean(xc * xc, axis=1, keepdims=True)
    xn = xc * lax.rsqrt(var + 1e-5)
    x = xn * g_ref[...] + be_ref[...]
    o_ref[...] = 0.5 * x * (1.0 + lax.erf(x * 0.7071067811865476))


def _dense_call(agg_t, h, W, b, gamma, beta):
    return pl.pallas_call(
        _dense_kernel,
        grid=(NP // BN,),
        in_specs=[
            pl.BlockSpec((D, BN), lambda i: (0, i)),
            pl.BlockSpec((BN, D), lambda i: (i, 0)),
            pl.BlockSpec((D, D), lambda i: (0, 0)),
            pl.BlockSpec((1, D), lambda i: (0, 0)),
            pl.BlockSpec((1, D), lambda i: (0, 0)),
            pl.BlockSpec((1, D), lambda i: (0, 0)),
        ],
        out_specs=pl.BlockSpec((BN, D), lambda i: (i, 0)),
        out_shape=jax.ShapeDtypeStruct((NP, D), jnp.float32),
    )(agg_t, h, W, b, gamma, beta)


def kernel(h, edge_index, edge_weight, W, b, gamma, beta):
    src = edge_index[0].astype(jnp.int32)
    dst = edge_index[1].astype(jnp.int32)
    ew = edge_weight.astype(jnp.float32)
    h = h.astype(jnp.float32)
    h_p = jnp.pad(h, ((0, NP - N), (0, 0)))

    hb = lax.bitcast_convert_type(
        jnp.pad(h.T, ((0, 0), (0, NP - N))).astype(jnp.bfloat16), jnp.uint16
    ).astype(jnp.uint32)
    hpk = lax.bitcast_convert_type(hb[0::2] | (hb[1::2] << 16), jnp.int32)

    spk = src | (dst << 14)

    degp = _deg_call(dst.reshape(NW, E_PER_W), ew.reshape(NW, E_PER_W))
    norm, d2 = _norm_call(degp, spk, ew)
    agg_t = _agg_call(hpk.reshape(-1), spk, norm, d2).reshape(D, NP)
    out = _dense_call(agg_t, h_p, W, b.reshape(1, D), gamma.reshape(1, D),
                      beta.reshape(1, D))
    return out[:N]
